# baseline (speedup 1.0000x reference)
import jax
import jax.numpy as jnp
from jax import lax
from jax.experimental import pallas as pl
from jax.experimental.pallas import tpu as pltpu

_NCH = 8
_NSLOT = 13


def _make_body(n_rows, n_cols, ncores):
    ngroups = n_rows // 8
    ngl = ngroups // ncores
    njl = ngl * _NCH
    nsteps = njl * 3
    step_w = (n_cols // _NCH) // 128 * 128
    tail = n_cols - _NCH * step_w
    last = _NCH - 1

    def body(x_hbm, o_hbm, buf, macc, sacc, isems, osems, itsems, otsems):
        core = lax.axis_index("core") if ncores > 1 else 0
        goff = core * ngl

        def _gc(jl):
            if isinstance(jl, int):
                return jl // _NCH, jl % _NCH
            return lax.div(jl, _NCH), lax.rem(jl, _NCH)

        def in_main(jl, sl):
            gl, cc = _gc(jl)
            return pltpu.make_async_copy(
                x_hbm.at[pl.ds((goff + gl) * 8, 8), pl.ds(cc * step_w, step_w)],
                buf.at[sl, :, pl.ds(0, step_w)],
                isems.at[sl],
            )

        def in_tail(jl, sl):
            gl, _ = _gc(jl)
            return pltpu.make_async_copy(
                x_hbm.at[pl.ds((goff + gl) * 8, 8), pl.ds(_NCH * step_w, tail)],
                buf.at[sl, :, pl.ds(step_w, tail)],
                itsems.at[sl],
            )

        def out_main(jl, sl):
            gl, cc = _gc(jl)
            return pltpu.make_async_copy(
                buf.at[sl, :, pl.ds(0, step_w)],
                o_hbm.at[pl.ds((goff + gl) * 8, 8), pl.ds(cc * step_w, step_w)],
                osems.at[sl],
            )

        def out_tail(jl, sl):
            gl, _ = _gc(jl)
            return pltpu.make_async_copy(
                buf.at[sl, :, pl.ds(step_w, tail)],
                o_hbm.at[pl.ds((goff + gl) * 8, 8), pl.ds(_NCH * step_w, tail)],
                otsems.at[sl],
            )

        def start_in(jl, sl, is_last):
            in_main(jl, sl).start()

            @pl.when(is_last)
            def _():
                in_tail(jl, sl).start()

        for jj in range(min(_NSLOT, njl)):
            start_in(jj, jj % _NSLOT, jj % _NCH == last)

        def step_fn(s, carry):
            gl = lax.div(s, 3 * _NCH)
            rem = lax.rem(s, 3 * _NCH)
            p = lax.div(rem, _NCH)
            c = lax.rem(rem, _NCH)
            jl = gl * _NCH + c
            slot = lax.rem(jl, _NSLOT)

            @pl.when(p == 0)
            def _():
                in_main(jl, slot).wait()

                @pl.when(c == last)
                def _():
                    in_tail(jl, slot).wait()

                x = buf[slot]
                mc = jnp.max(x[:, :step_w], axis=1, keepdims=True)
                mt = jnp.max(x[:, step_w:], axis=1, keepdims=True)
                mc = jnp.maximum(mc, jnp.where(c == last, mt, -jnp.inf))
                macc[...] = jnp.where(c == 0, mc, jnp.maximum(macc[...], mc))

            @pl.when(p == 1)
            def _():
                x = buf[slot]
                e = jnp.exp(x - macc[...])
                sc_main = jnp.sum(e[:, :step_w], axis=1, keepdims=True)
                sc_tail = jnp.sum(e[:, step_w:], axis=1, keepdims=True)
                sc = sc_main + jnp.where(c == last, sc_tail, 0.0)
                sacc[...] = jnp.where(c == 0, sc, sacc[...] + sc)

            @pl.when(p == 2)
            def _():
                x = buf[slot]
                buf[slot] = jnp.exp(x - macc[...]) * (1.0 / sacc[...])
                out_main(jl, slot).start()

                @pl.when(c == last)
                def _():
                    out_tail(jl, slot).start()

                @pl.when(jl >= 5)
                def _():
                    jprev = jl - 5
                    slprev = lax.rem(jprev, _NSLOT)
                    _, ccprev = _gc(jprev)
                    out_main(jprev, slprev).wait()

                    @pl.when(ccprev == last)
                    def _():
                        out_tail(jprev, slprev).wait()

                    @pl.when(jl + 8 < njl)
                    def _():
                        _, ccnext = _gc(jl + 8)
                        start_in(jl + 8, slprev, ccnext == last)

            return carry

        lax.fori_loop(0, nsteps, step_fn, 0)

        for dj in range(max(njl - 5, 0), njl):
            out_main(dj, dj % _NSLOT).wait()
            if dj % _NCH == last:
                out_tail(dj, dj % _NSLOT).wait()

    return body


def kernel(logits):
    n_rows, n_cols = logits.shape
    try:
        ncores = jax.devices()[0].num_cores
    except Exception:
        ncores = 1
    ngroups = n_rows // 8
    if ngroups % ncores != 0:
        ncores = 1
    step_w = (n_cols // _NCH) // 128 * 128
    csize = n_cols - (_NCH - 1) * step_w
    mesh = pltpu.create_tensorcore_mesh("core", num_cores=ncores)
    body = _make_body(n_rows, n_cols, ncores)
    f = pl.kernel(
        body,
        out_type=jax.ShapeDtypeStruct((n_rows, n_cols), logits.dtype),
        mesh=mesh,
        scratch_types=[
            pltpu.VMEM((_NSLOT, 8, csize), jnp.float32),
            pltpu.VMEM((8, 1), jnp.float32),
            pltpu.VMEM((8, 1), jnp.float32),
            pltpu.SemaphoreType.DMA((_NSLOT,)),
            pltpu.SemaphoreType.DMA((_NSLOT,)),
            pltpu.SemaphoreType.DMA((_NSLOT,)),
            pltpu.SemaphoreType.DMA((_NSLOT,)),
        ],
    )
    return f(logits)


# speedup vs baseline: 1.0564x; 1.0564x over previous
import jax
import jax.numpy as jnp
from jax import lax
from jax.experimental import pallas as pl
from jax.experimental.pallas import tpu as pltpu

_NCH = 8
_NSLOT = 13


def _make_body(n_rows, n_cols, ncores):
    ngroups = n_rows // 8
    ngl = ngroups // ncores
    njl = ngl * _NCH
    nsteps = njl * 3
    step_w = (n_cols // _NCH) // 128 * 128
    tail = n_cols - _NCH * step_w
    last = _NCH - 1

    def body(x_hbm, o_hbm, buf, macc, sacc, isems, osems, itsems, otsems):
        core = lax.axis_index("core") if ncores > 1 else 0
        goff = core * ngl

        def _gc(jl):
            if isinstance(jl, int):
                return jl // _NCH, jl % _NCH
            return lax.div(jl, _NCH), lax.rem(jl, _NCH)

        def in_main(jl, sl):
            gl, cc = _gc(jl)
            return pltpu.make_async_copy(
                x_hbm.at[pl.ds((goff + gl) * 8, 8), pl.ds(cc * step_w, step_w)],
                buf.at[sl, :, pl.ds(0, step_w)],
                isems.at[sl],
            )

        def in_tail(jl, sl):
            gl, _ = _gc(jl)
            return pltpu.make_async_copy(
                x_hbm.at[pl.ds((goff + gl) * 8, 8), pl.ds(_NCH * step_w, tail)],
                buf.at[sl, :, pl.ds(step_w, tail)],
                itsems.at[sl],
            )

        def out_main(jl, sl):
            gl, cc = _gc(jl)
            return pltpu.make_async_copy(
                buf.at[sl, :, pl.ds(0, step_w)],
                o_hbm.at[pl.ds((goff + gl) * 8, 8), pl.ds(cc * step_w, step_w)],
                osems.at[sl],
            )

        def out_tail(jl, sl):
            gl, _ = _gc(jl)
            return pltpu.make_async_copy(
                buf.at[sl, :, pl.ds(step_w, tail)],
                o_hbm.at[pl.ds((goff + gl) * 8, 8), pl.ds(_NCH * step_w, tail)],
                otsems.at[sl],
            )

        def start_in(jl, sl, is_last):
            in_main(jl, sl).start()

            @pl.when(is_last)
            def _():
                in_tail(jl, sl).start()

        for jj in range(min(_NSLOT, njl)):
            start_in(jj, jj % _NSLOT, jj % _NCH == last)

        def step_fn(s, carry):
            gl = lax.div(s, 3 * _NCH)
            rem = lax.rem(s, 3 * _NCH)
            p = lax.div(rem, _NCH)
            c = lax.rem(rem, _NCH)
            jl = gl * _NCH + c
            slot = lax.rem(jl, _NSLOT)

            @pl.when(p == 0)
            def _():
                in_main(jl, slot).wait()

                @pl.when(c == last)
                def _():
                    in_tail(jl, slot).wait()

                x = buf[slot]
                mc = jnp.max(x[:, :step_w], axis=1, keepdims=True)
                mt = jnp.max(x[:, step_w:], axis=1, keepdims=True)
                mc = jnp.maximum(mc, jnp.where(c == last, mt, -jnp.inf))
                macc[...] = jnp.where(c == 0, mc, jnp.maximum(macc[...], mc))

            @pl.when(p == 1)
            def _():
                x = buf[slot]
                e = jnp.exp(x - macc[...])
                sc_main = jnp.sum(e[:, :step_w], axis=1, keepdims=True)
                sc_tail = jnp.sum(e[:, step_w:], axis=1, keepdims=True)
                sc = sc_main + jnp.where(c == last, sc_tail, 0.0)
                sacc[...] = jnp.where(c == 0, sc, sacc[...] + sc)

            @pl.when(p == 2)
            def _():
                x = buf[slot]
                buf[slot] = jnp.exp(x - macc[...]) * (1.0 / sacc[...])
                out_main(jl, slot).start()

                @pl.when(c == last)
                def _():
                    out_tail(jl, slot).start()

                @pl.when(jl >= 5)
                def _():
                    jprev = jl - 5
                    slprev = lax.rem(jprev, _NSLOT)
                    _, ccprev = _gc(jprev)
                    out_main(jprev, slprev).wait()

                    @pl.when(ccprev == last)
                    def _():
                        out_tail(jprev, slprev).wait()

                    @pl.when(jl + 8 < njl)
                    def _():
                        _, ccnext = _gc(jl + 8)
                        start_in(jl + 8, slprev, ccnext == last)

            return carry

        lax.fori_loop(0, nsteps, step_fn, 0)

        for dj in range(max(njl - 5, 0), njl):
            out_main(dj, dj % _NSLOT).wait()
            if dj % _NCH == last:
                out_tail(dj, dj % _NSLOT).wait()

    return body


def kernel(logits):
    n_rows, n_cols = logits.shape
    ncores = 2
    ngroups = n_rows // 8
    if ngroups % ncores != 0:
        ncores = 1
    step_w = (n_cols // _NCH) // 128 * 128
    csize = n_cols - (_NCH - 1) * step_w
    mesh = pltpu.create_tensorcore_mesh("core", num_cores=ncores)
    body = _make_body(n_rows, n_cols, ncores)
    f = pl.kernel(
        body,
        out_type=jax.ShapeDtypeStruct((n_rows, n_cols), logits.dtype),
        mesh=mesh,
        scratch_types=[
            pltpu.VMEM((_NSLOT, 8, csize), jnp.float32),
            pltpu.VMEM((8, 1), jnp.float32),
            pltpu.VMEM((8, 1), jnp.float32),
            pltpu.SemaphoreType.DMA((_NSLOT,)),
            pltpu.SemaphoreType.DMA((_NSLOT,)),
            pltpu.SemaphoreType.DMA((_NSLOT,)),
            pltpu.SemaphoreType.DMA((_NSLOT,)),
        ],
    )
    return f(logits)
